# Initial kernel scaffold; baseline (speedup 1.0000x reference)
import jax
import jax.numpy as jnp
from jax.experimental import pallas as pl


def kernel(atom_features, r, edge_index, Wa, ba, ga, bba, We1, be1, g1, bb1, We2, be2, g2, bb2, gcn_weight, gcn_bias, srcW, srcb, dstW, dstb, edgW, edgb, W1, b1, W2, b2, W3, b3):
    raise NotImplementedError("write your pallas kernel here")


# speedup vs baseline: 1.0590x; 1.0590x over previous
import functools

import numpy as np
import jax
import jax.numpy as jnp
from jax import lax
from jax.experimental import pallas as pl
from jax.experimental.pallas import tpu as pltpu
from jax.experimental.pallas import tpu_sc as plsc

N = 10000
E = 320000
D_ATOM = 92
BINS = 80
EMB = 64
H = 128
L = 4
FMAP = 10000

_EPS = 1e-5
_CENTERS = np.linspace(0.0, 8.0, BINS, dtype=np.float32)
_GAMMA = np.float32(1.0 / np.diff(_CENTERS).mean())
_CSTEP = np.float32(8.0 / (BINS - 1))

_NC = 2
_NS = 16
_NW = _NC * _NS
_EPW = E // _NW
_R = 40
_NCHUNK = _EPW // _R


def _atom_body(af_ref, wa_ref, ba_ref, ga_ref, bba_ref, out_ref):
    z = jnp.dot(af_ref[...], wa_ref[...],
                preferred_element_type=jnp.float32) + ba_ref[...]
    mu = jnp.mean(z, axis=0, keepdims=True)
    var = jnp.mean((z - mu) ** 2, axis=0, keepdims=True)
    xn = ga_ref[...] * (z - mu) * lax.rsqrt(var + _EPS) + bba_ref[...]
    out_ref[...] = jnp.maximum(xn, 0.0)


def _atom_embed(af, Wa, ba, ga, bba):
    return pl.pallas_call(
        _atom_body,
        out_shape=jax.ShapeDtypeStruct((N, H), jnp.float32),
    )(af, Wa, ba.reshape(1, H), ga.reshape(1, H), bba.reshape(1, H))


_BE = 4000


def _edge1_body(r_ref, we1_ref, be1_ref, z1_ref, st_ref):
    r = r_ref[...]
    bond = jnp.sqrt(jnp.sum(r * r, axis=1, keepdims=True))
    centers = lax.broadcasted_iota(
        jnp.int32, (1, BINS), 1).astype(jnp.float32) * _CSTEP
    rbf = jnp.exp(-_GAMMA * (bond - centers) ** 2)
    z1 = jnp.dot(rbf, we1_ref[...],
                 preferred_element_type=jnp.float32) + be1_ref[...]
    z1_ref[...] = z1
    blk = jnp.concatenate(
        [jnp.sum(z1, axis=0, keepdims=True),
         jnp.sum(z1 * z1, axis=0, keepdims=True),
         jnp.zeros((6, EMB), jnp.float32)], axis=0)

    @pl.when(pl.program_id(0) == 0)
    def _():
        st_ref[...] = blk

    @pl.when(pl.program_id(0) > 0)
    def _():
        st_ref[...] += blk


def _edge1(r, We1, be1):
    return pl.pallas_call(
        _edge1_body,
        grid=(E // _BE,),
        in_specs=[
            pl.BlockSpec((_BE, 3), lambda i: (i, 0)),
            pl.BlockSpec((BINS, EMB), lambda i: (0, 0)),
            pl.BlockSpec((1, EMB), lambda i: (0, 0)),
        ],
        out_specs=[
            pl.BlockSpec((_BE, EMB), lambda i: (i, 0)),
            pl.BlockSpec((8, EMB), lambda i: (0, 0)),
        ],
        out_shape=[
            jax.ShapeDtypeStruct((E, EMB), jnp.float32),
            jax.ShapeDtypeStruct((8, EMB), jnp.float32),
        ],
    )(r, We1, be1.reshape(1, EMB))


def _affine_from_stats(st, g, b):
    mu = st[0:1] / E
    var = st[1:2] / E - mu * mu
    scale = g * lax.rsqrt(var + _EPS)
    shift = b - mu * scale
    return scale, shift


def _edge2_body(z1_ref, st1_ref, g1_ref, bb1_ref, we2_ref, be2_ref,
                z2_ref, st_ref):
    scale, shift = _affine_from_stats(st1_ref[...], g1_ref[...], bb1_ref[...])
    y1 = jnp.maximum(z1_ref[...] * scale + shift, 0.0)
    z2 = jnp.dot(y1, we2_ref[...],
                 preferred_element_type=jnp.float32) + be2_ref[...]
    z2_ref[...] = z2
    blk = jnp.concatenate(
        [jnp.sum(z2, axis=0, keepdims=True),
         jnp.sum(z2 * z2, axis=0, keepdims=True),
         jnp.zeros((6, H), jnp.float32)], axis=0)

    @pl.when(pl.program_id(0) == 0)
    def _():
        st_ref[...] = blk

    @pl.when(pl.program_id(0) > 0)
    def _():
        st_ref[...] += blk


def _edge2(z1, st1, g1, bb1, We2, be2):
    return pl.pallas_call(
        _edge2_body,
        grid=(E // _BE,),
        in_specs=[
            pl.BlockSpec((_BE, EMB), lambda i: (i, 0)),
            pl.BlockSpec((8, EMB), lambda i: (0, 0)),
            pl.BlockSpec((1, EMB), lambda i: (0, 0)),
            pl.BlockSpec((1, EMB), lambda i: (0, 0)),
            pl.BlockSpec((EMB, H), lambda i: (0, 0)),
            pl.BlockSpec((1, H), lambda i: (0, 0)),
        ],
        out_specs=[
            pl.BlockSpec((_BE, H), lambda i: (i, 0)),
            pl.BlockSpec((8, H), lambda i: (0, 0)),
        ],
        out_shape=[
            jax.ShapeDtypeStruct((E, H), jnp.float32),
            jax.ShapeDtypeStruct((8, H), jnp.float32),
        ],
    )(z1, st1, g1.reshape(1, EMB), bb1.reshape(1, EMB), We2,
      be2.reshape(1, H))


def _ee_body(z2_ref, st2_ref, g2_ref, bb2_ref, wcat_ref, bcat_ref,
             e0_ref, e1_ref, e2_ref, e3_ref):
    scale, shift = _affine_from_stats(st2_ref[...], g2_ref[...], bb2_ref[...])
    y = jnp.maximum(z2_ref[...] * scale + shift, 0.0)
    ee = jnp.dot(y, wcat_ref[...],
                 preferred_element_type=jnp.float32) + bcat_ref[...]
    e0_ref[...] = ee[:, 0:H]
    e1_ref[...] = ee[:, H:2 * H]
    e2_ref[...] = ee[:, 2 * H:3 * H]
    e3_ref[...] = ee[:, 3 * H:4 * H]


def _edge_gates(z2, st2, g2, bb2, edgW, edgb):
    wcat = jnp.transpose(edgW, (1, 0, 2)).reshape(H, L * H)
    bcat = edgb.reshape(1, L * H)
    return pl.pallas_call(
        _ee_body,
        grid=(E // _BE,),
        in_specs=[
            pl.BlockSpec((_BE, H), lambda i: (i, 0)),
            pl.BlockSpec((8, H), lambda i: (0, 0)),
            pl.BlockSpec((1, H), lambda i: (0, 0)),
            pl.BlockSpec((1, H), lambda i: (0, 0)),
            pl.BlockSpec((H, L * H), lambda i: (0, 0)),
            pl.BlockSpec((1, L * H), lambda i: (0, 0)),
        ],
        out_specs=[pl.BlockSpec((_BE, H), lambda i: (i, 0))] * L,
        out_shape=[jax.ShapeDtypeStruct((E, H), jnp.float32)] * L,
    )(z2, st2, g2.reshape(1, H), bb2.reshape(1, H), wcat, bcat)


def _nodes_body(x_ref, sw_ref, sb_ref, dw_ref, db_ref, d0_ref, d1_ref,
                comb_ref, ed_ref):
    x = x_ref[...]
    deg = jnp.maximum(d0_ref[...] + d1_ref[...], 1.0)
    norm_l = lax.rsqrt(deg)
    es = jnp.dot(x, sw_ref[...],
                 preferred_element_type=jnp.float32) + sb_ref[...]
    comb_ref[...] = jnp.concatenate([es, x * norm_l], axis=1)
    ed_ref[...] = jnp.dot(x, dw_ref[...],
                          preferred_element_type=jnp.float32) + db_ref[...]


def _node_tables(x, sw, sb, dw, db, dout0, dout1):
    return pl.pallas_call(
        _nodes_body,
        out_shape=[
            jax.ShapeDtypeStruct((N, 2 * H), jnp.float32),
            jax.ShapeDtypeStruct((N, H), jnp.float32),
        ],
    )(x, sw, sb.reshape(1, H), dw, db.reshape(1, H), dout0, dout1)


def _rst_body(x_ref, rst_ref, w_ref, b_ref, di0_ref, di1_ref, out_ref):
    rst = rst_ref[0] + rst_ref[1]
    deg = jnp.maximum(di0_ref[...] + di1_ref[...], 1.0)
    norm_r = lax.rsqrt(deg)
    t = jnp.dot(rst, w_ref[...], preferred_element_type=jnp.float32)
    out_ref[...] = x_ref[...] + t * norm_r + b_ref[...]


def _apply_rst(x, rst2, w, b, din0, din1):
    return pl.pallas_call(
        _rst_body,
        out_shape=jax.ShapeDtypeStruct((N, H), jnp.float32),
    )(x, rst2, w, b.reshape(1, H), din0, din1)


def _mlp1_body(x_ref, w1_ref, b1_ref, w2_ref, b2_ref, h2_ref):
    h1 = jnp.maximum(
        jnp.dot(x_ref[...], w1_ref[...],
                preferred_element_type=jnp.float32) + b1_ref[...], 0.0)
    h2_ref[...] = jnp.maximum(
        jnp.dot(h1, w2_ref[...],
                preferred_element_type=jnp.float32) + b2_ref[...], 0.0)


def _mlp1(x, W1, b1, W2, b2):
    BR = 2000
    return pl.pallas_call(
        _mlp1_body,
        grid=(N // BR,),
        in_specs=[
            pl.BlockSpec((BR, H), lambda i: (i, 0)),
            pl.BlockSpec((H, 4 * H), lambda i: (0, 0)),
            pl.BlockSpec((1, 4 * H), lambda i: (0, 0)),
            pl.BlockSpec((4 * H, 8 * H), lambda i: (0, 0)),
            pl.BlockSpec((1, 8 * H), lambda i: (0, 0)),
        ],
        out_specs=pl.BlockSpec((BR, 8 * H), lambda i: (i, 0)),
        out_shape=jax.ShapeDtypeStruct((N, 8 * H), jnp.float32),
    )(x, W1, b1.reshape(1, 4 * H), W2, b2.reshape(1, 8 * H))


def _mlp2_body(h2_ref, w3_ref, b3_ref, out_ref):
    out_ref[...] = jnp.maximum(
        jnp.dot(h2_ref[...], w3_ref[...],
                preferred_element_type=jnp.float32) + b3_ref[...], 0.0)


def _mlp2(h2, W3, b3):
    BR, BC = 2000, 1024
    nc = pl.cdiv(FMAP, BC)
    return pl.pallas_call(
        _mlp2_body,
        grid=(nc, N // BR),
        in_specs=[
            pl.BlockSpec((BR, 8 * H), lambda j, i: (i, 0)),
            pl.BlockSpec((8 * H, BC), lambda j, i: (0, j)),
            pl.BlockSpec((1, BC), lambda j, i: (0, j)),
        ],
        out_specs=pl.BlockSpec((BR, BC), lambda j, i: (i, j)),
        out_shape=jax.ShapeDtypeStruct((N, FMAP), jnp.float32),
    )(h2, W3, b3.reshape(1, FMAP))


def _sc_degrees(src, dst, zeros1, ones1):
    mesh = plsc.VectorSubcoreMesh(core_axis_name="c", subcore_axis_name="s")

    @functools.partial(
        pl.kernel, mesh=mesh,
        out_type=[jax.ShapeDtypeStruct((N,), jnp.float32)] * 4,
        scratch_types=[
            pltpu.VMEM((_EPW,), jnp.int32),
            pltpu.VMEM((_EPW,), jnp.float32),
            pltpu.VMEM_SHARED((N,), jnp.float32),
            pltpu.VMEM_SHARED((N,), jnp.float32),
        ],
    )
    def deg_kernel(src_hbm, dst_hbm, z_hbm, o_hbm, out0, out1, out2, out3,
                   idx_v, ones_v, acc_out, acc_in):
        c = lax.axis_index("c")
        s = lax.axis_index("s")

        @pl.when(s == 0)
        def _():
            pltpu.sync_copy(z_hbm, acc_out)
            pltpu.sync_copy(z_hbm, acc_in)

        plsc.subcore_barrier()
        pltpu.sync_copy(o_hbm, ones_v)
        base = (c * _NS + s) * _EPW
        pltpu.sync_copy(src_hbm.at[pl.ds(base, _EPW)], idx_v)
        pltpu.sync_copy(ones_v, acc_out.at[idx_v], add=True)
        pltpu.sync_copy(dst_hbm.at[pl.ds(base, _EPW)], idx_v)
        pltpu.sync_copy(ones_v, acc_in.at[idx_v], add=True)
        plsc.subcore_barrier()

        @pl.when((s == 0) & (c == 0))
        def _():
            pltpu.sync_copy(acc_out, out0)
            pltpu.sync_copy(acc_in, out1)

        @pl.when((s == 0) & (c == 1))
        def _():
            pltpu.sync_copy(acc_out, out2)
            pltpu.sync_copy(acc_in, out3)

    return deg_kernel(src, dst, zeros1, ones1)


def _sc_layer(src, dst, comb, ed, ee, zeros2):
    mesh = plsc.VectorSubcoreMesh(core_axis_name="c", subcore_axis_name="s")
    rpc = 1000

    @functools.partial(
        pl.kernel, mesh=mesh,
        out_type=jax.ShapeDtypeStruct((2, N, H), jnp.float32),
        scratch_types=[
            pltpu.VMEM((_R,), jnp.int32),
            pltpu.VMEM((_R,), jnp.int32),
            pltpu.VMEM((_R, 2 * H), jnp.float32),
            pltpu.VMEM((_R, H), jnp.float32),
            pltpu.VMEM((_R, H), jnp.float32),
            pltpu.VMEM((_R, H), jnp.float32),
            pltpu.VMEM_SHARED((N, H), jnp.float32),
            pltpu.SemaphoreType.DMA,
            pltpu.SemaphoreType.DMA,
            pltpu.SemaphoreType.DMA,
        ],
    )
    def layer_kernel(src_hbm, dst_hbm, comb_hbm, ed_hbm, ee_hbm, z_hbm,
                     out_hbm, idx_s, idx_d, comb_g, ed_g, ee_v, msg_v, acc,
                     sem0, sem1, sem2):
        c = lax.axis_index("c")
        s = lax.axis_index("s")

        @pl.when(s == 0)
        def _():
            pltpu.sync_copy(z_hbm, acc)

        plsc.subcore_barrier()

        wbase = (c * _NS + s) * _EPW

        def chunk(t, carry):
            base = wbase + t * _R
            pltpu.sync_copy(src_hbm.at[pl.ds(base, _R)], idx_s)
            pltpu.sync_copy(dst_hbm.at[pl.ds(base, _R)], idx_d)
            cp0 = pltpu.async_copy(comb_hbm.at[idx_s], comb_g, sem0)
            cp1 = pltpu.async_copy(ed_hbm.at[idx_d], ed_g, sem1)
            cp2 = pltpu.async_copy(ee_hbm.at[pl.ds(base, _R)], ee_v, sem2)
            cp0.wait()
            cp1.wait()
            cp2.wait()

            def row(rr, carry2):
                for j in range(H // 16):
                    sl = pl.ds(j * 16, 16)
                    v = (comb_g[rr, sl] + ed_g[rr, sl] + ee_v[rr, sl])
                    sig = 1.0 / (1.0 + jnp.exp(-v))
                    msg_v[rr, sl] = comb_g[rr, pl.ds(H + j * 16, 16)] * sig
                return carry2

            lax.fori_loop(0, _R, row, 0)
            pltpu.sync_copy(msg_v, acc.at[idx_d], add=True)
            return carry

        lax.fori_loop(0, _NCHUNK, chunk, 0)
        plsc.subcore_barrier()

        @pl.when(s < 10)
        def _():
            pltpu.sync_copy(acc.at[pl.ds(s * rpc, rpc)],
                            out_hbm.at[c, pl.ds(s * rpc, rpc)])

    return layer_kernel(src, dst, comb, ed, ee, zeros2)


def kernel(atom_features, r, edge_index, Wa, ba, ga, bba, We1, be1, g1, bb1,
           We2, be2, g2, bb2, gcn_weight, gcn_bias, srcW, srcb, dstW, dstb,
           edgW, edgb, W1, b1, W2, b2, W3, b3):
    src = edge_index[0]
    dst = edge_index[1]

    zeros1 = jnp.zeros((N,), jnp.float32)
    ones1 = jnp.ones((_EPW,), jnp.float32)
    zeros2 = jnp.zeros((N, H), jnp.float32)

    x = _atom_embed(atom_features, Wa, ba, ga, bba)

    dout0, din0, dout1, din1 = _sc_degrees(src, dst, zeros1, ones1)
    dout0 = dout0.reshape(N, 1)
    dout1 = dout1.reshape(N, 1)
    din0 = din0.reshape(N, 1)
    din1 = din1.reshape(N, 1)

    z1, st1 = _edge1(r, We1, be1)
    z2, st2 = _edge2(z1, st1, g1, bb1, We2, be2)
    ees = _edge_gates(z2, st2, g2, bb2, edgW, edgb)

    for i in range(L):
        comb, ed = _node_tables(x, srcW[i], srcb[i], dstW[i], dstb[i],
                                dout0, dout1)
        rst2 = _sc_layer(src, dst, comb, ed, ees[i], zeros2)
        x = _apply_rst(x, rst2, gcn_weight[i], gcn_bias[i], din0, din1)

    h2 = _mlp1(x, W1, b1, W2, b2)
    out = _mlp2(h2, W3, b3)
    return out.reshape(-1, 100, 100)
